# baseline (speedup 1.0000x reference)
import functools

import jax
import jax.numpy as jnp
from jax.experimental import pallas as pl
from jax.experimental.pallas import tpu as pltpu

_V = 30000


def _round_up(x, m):
    return ((x + m - 1) // m) * m


def _pool_head_kernel(idx_ref, emb_ref, w_ref, b_ref, o_ref, pooled_ref,
                      *, tb, s):
    row0 = pl.program_id(0) * tb
    for r in range(tb):
        acc_a, acc_b = None, None
        for t in range(s):
            slab = emb_ref[pl.ds(pl.multiple_of(idx_ref[row0 + r, t], 2), 2), :]
            if t % 2 == 0:
                acc_a = slab if acc_a is None else acc_a + slab
            else:
                acc_b = slab if acc_b is None else acc_b + slab
        acc2 = acc_a if acc_b is None else acc_a + acc_b
        pooled_ref[pl.ds(r, 1), :] = jnp.concatenate(
            [acc2[0:1, :], acc2[1:2, :]], axis=1)
    logits = jnp.dot(pooled_ref[...], w_ref[...],
                     preferred_element_type=jnp.float32)
    o_ref[...] = logits + b_ref[...]


def kernel(x, emb, w, b):
    B, S = x.shape
    Vr, Hp = emb.shape
    Cp = w.shape[1]
    tb = 64
    Bp = _round_up(B, tb)

    ids = x.astype(jnp.int32)
    ids = jnp.where((ids >= 0) & (ids < _V), ids, _V)
    ids = jnp.pad(ids, ((0, Bp - B), (0, 0)), constant_values=_V)
    ids2 = ids * jnp.int32(2)

    emb2 = emb.reshape(2 * Vr, Hp // 2)
    w_scaled = w * jnp.float32(1.0 / S)

    out = pl.pallas_call(
        functools.partial(_pool_head_kernel, tb=tb, s=S),
        out_shape=jax.ShapeDtypeStruct((Bp, Cp), jnp.float32),
        grid_spec=pltpu.PrefetchScalarGridSpec(
            num_scalar_prefetch=1,
            grid=(Bp // tb,),
            in_specs=[
                pl.BlockSpec((2 * Vr, Hp // 2), lambda i, idx: (0, 0),
                             pipeline_mode=pl.Buffered(1)),
                pl.BlockSpec((Hp, Cp), lambda i, idx: (0, 0),
                             pipeline_mode=pl.Buffered(1)),
                pl.BlockSpec((1, Cp), lambda i, idx: (0, 0),
                             pipeline_mode=pl.Buffered(1)),
            ],
            out_specs=pl.BlockSpec((tb, Cp), lambda i, idx: (i, 0)),
            scratch_shapes=[pltpu.VMEM((tb, Hp), jnp.float32)],
        ),
        compiler_params=pltpu.CompilerParams(
            dimension_semantics=("parallel",),
            vmem_limit_bytes=48 * 1024 * 1024,
        ),
    )(ids2, emb2, w_scaled, b)
    return out[:B, :Cp]
